# baseline (speedup 1.0000x reference)
import functools

import jax
import jax.numpy as jnp
from jax import lax
from jax.experimental import pallas as pl
from jax.experimental.pallas import tpu as pltpu
from jax.experimental.pallas import tpu_sc as plsc

N = 10000
D = 128
H = 128
E_UP = 320000
E_DOWN = 160000
E_B = 160000

NC = 2
NS = 16
L = 16
C = 40
RPT = 624
RTAIL = N - RPT * NS


def _prep_body(x_ref, au_ref, bu_ref, ad_ref, bd_ref, yu_ref, yd_ref):
    x = x_ref[...]
    yu_ref[...] = jnp.dot(x, au_ref[...], preferred_element_type=jnp.float32) + bu_ref[...]
    yd_ref[...] = jnp.dot(x, ad_ref[...], preferred_element_type=jnp.float32) + bd_ref[...]


def _prep(x, au, bu, ad, bd):
    return pl.pallas_call(
        _prep_body,
        out_shape=[jax.ShapeDtypeStruct((N, H), jnp.float32)] * 2,
    )(x, au, bu.reshape(1, H), ad, bd.reshape(1, H))


def _t_body(attr_ref, w_ref, t_ref):
    t_ref[...] = jnp.dot(attr_ref[...], w_ref[...], preferred_element_type=jnp.float32)


def _edge_matmul(attr, w, block=1280):
    e = attr.shape[0]
    return pl.pallas_call(
        _t_body,
        grid=(e // block,),
        in_specs=[
            pl.BlockSpec((block, D), lambda i: (i, 0)),
            pl.BlockSpec((D, H), lambda i: (0, 0)),
        ],
        out_specs=pl.BlockSpec((block, H), lambda i: (i, 0)),
        out_shape=jax.ShapeDtypeStruct((e, H), jnp.float32),
    )(attr, w)


def _sc_body(yu, tu, u0, u1, yd, td, d0, d1, x, b0, b1,
             out_up, out_dn, out_b,
             acc, tb, gb, i0b, i1b, in_sem, g_sem, sc_sem, i1_sem):
    c = lax.axis_index("c")
    s = lax.axis_index("s")

    def rows_copy(src, dst):
        r = pl.ds(pl.multiple_of(s * RPT, 8), RPT)
        pltpu.sync_copy(src.at[r], dst.at[r])

        @pl.when(s == 0)
        def _():
            rt = pl.ds(RPT * NS, RTAIL)
            pltpu.sync_copy(src.at[rt], dst.at[rt])

    def init_acc():
        rows_copy(x, acc)

    def edge_loop(y_ref, t_ref, i0_ref, i1_ref, edges_per_tile, with_t):
        n = edges_per_tile // C
        tile_base = s * edges_per_tile

        def issue_in(jj, s4):
            b = pl.multiple_of(tile_base + jj * C, 8)
            pltpu.async_copy(i0_ref.at[pl.ds(b, C)], i0b.at[s4], in_sem.at[s4])
            if with_t:
                pltpu.async_copy(t_ref.at[pl.ds(b, C)], tb.at[s4], in_sem.at[s4])

        def wait_in(s4):
            pltpu.make_async_copy(i0_ref.at[pl.ds(0, C)], i0b.at[s4], in_sem.at[s4]).wait()
            if with_t:
                pltpu.make_async_copy(t_ref.at[pl.ds(0, C)], tb.at[s4], in_sem.at[s4]).wait()

        def chunk(j, s4, wait_sc, pre_in):
            if wait_sc:
                pltpu.make_async_copy(gb.at[s4], acc.at[i1b.at[s4]], sc_sem.at[s4]).wait()
            b = pl.multiple_of(tile_base + j * C, 8)
            pltpu.async_copy(i1_ref.at[pl.ds(b, C)], i1b.at[s4], i1_sem.at[s4])
            wait_in(s4)
            pltpu.async_copy(y_ref.at[i0b.at[s4]], gb.at[s4], g_sem.at[s4])
            if pre_in:
                issue_in(j + 2, (s4 + 2) % 4)
            pltpu.make_async_copy(y_ref.at[i0b.at[s4]], gb.at[s4], g_sem.at[s4]).wait()
            if with_t:
                gs = gb.at[s4]
                ts = tb.at[s4]

                def row(r, rcarry):
                    for cb in range(H // L):
                        sl = pl.ds(cb * L, L)
                        gs[r, sl] = jnp.maximum(gs[r, sl] + ts[r, sl], 0.0)
                    return rcarry

                lax.fori_loop(0, C, row, 0, unroll=2)
            pltpu.make_async_copy(i1_ref.at[pl.ds(0, C)], i1b.at[s4], i1_sem.at[s4]).wait()
            pltpu.async_copy(gb.at[s4], acc.at[i1b.at[s4]], sc_sem.at[s4], add=True)

        issue_in(0, 0)
        issue_in(1, 1)
        for j in range(4):
            chunk(j, j % 4, False, j + 2 < n)
        g4 = n // 4

        def group(jg, carry):
            for r in range(4):
                chunk(jg * 4 + r, r, True, True)
            return carry

        lax.fori_loop(1, g4 - 1, group, 0, unroll=False)
        for j in range(4 * (g4 - 1), n):
            chunk(j, j % 4, True, j + 2 < n)
        for j in range(n - 4, n):
            s4 = j % 4
            pltpu.make_async_copy(gb.at[s4], acc.at[i1b.at[s4]], sc_sem.at[s4]).wait()

    init_acc()
    plsc.subcore_barrier()

    @pl.when(c == 0)
    def _():
        edge_loop(yu, tu, u0, u1, E_UP // NS, True)

    @pl.when(c == 1)
    def _():
        edge_loop(yd, td, d0, d1, E_DOWN // NS, True)

    plsc.subcore_barrier()

    @pl.when(c == 0)
    def _():
        rows_copy(acc, out_up)

    @pl.when(c == 1)
    def _():
        rows_copy(acc, out_dn)

    plsc.subcore_barrier()

    @pl.when(c == 1)
    def _():
        init_acc()

    plsc.subcore_barrier()

    @pl.when(c == 1)
    def _():
        edge_loop(x, None, b0, b1, E_B // NS, False)

    plsc.subcore_barrier()

    @pl.when(c == 1)
    def _():
        rows_copy(acc, out_b)


def _sc_aggregate(yu, tu, u0, u1, yd, td, d0, d1, x, b0, b1):
    mesh = plsc.VectorSubcoreMesh(
        core_axis_name="c", subcore_axis_name="s", num_cores=NC, num_subcores=NS)
    return pl.kernel(
        _sc_body,
        out_type=[jax.ShapeDtypeStruct((N, H), jnp.float32)] * 3,
        mesh=mesh,
        scratch_types=[
            pltpu.VMEM_SHARED((N, H), jnp.float32),
            pltpu.VMEM((4, C, H), jnp.float32),
            pltpu.VMEM((4, C, H), jnp.float32),
            pltpu.VMEM((4, C), jnp.int32),
            pltpu.VMEM((4, C), jnp.int32),
            pltpu.SemaphoreType.DMA((4,)),
            pltpu.SemaphoreType.DMA((4,)),
            pltpu.SemaphoreType.DMA((4,)),
            pltpu.SemaphoreType.DMA((4,)),
        ],
    )(yu, tu, u0, u1, yd, td, d0, d1, x, b0, b1)


def _bn_relu(h, g, b):
    mu = jnp.mean(h, axis=0, keepdims=True)
    var = jnp.mean((h - mu) * (h - mu), axis=0, keepdims=True)
    return jnp.maximum((h - mu) * lax.rsqrt(var + 1e-5) * g + b, 0.0)


def _update_path(h, w1, b1, g1, be1, w2, b2, g2, be2):
    h = _bn_relu(jnp.dot(h, w1, preferred_element_type=jnp.float32) + b1, g1, be1)
    h = _bn_relu(jnp.dot(h, w2, preferred_element_type=jnp.float32) + b2, g2, be2)
    return h


def _final_body(au_ref, ad_ref, ab_ref, *refs):
    prefs = refs[:-1]
    o_ref = refs[-1]
    pu = [r[...] for r in prefs[0:8]]
    pd = [r[...] for r in prefs[8:16]]
    pb = [r[...] for r in prefs[16:24]]
    wc1, wc2, wc3, bc, gc, bec = [r[...] for r in prefs[24:30]]
    hu = _update_path(au_ref[...], *pu)
    hd = _update_path(ad_ref[...], *pd)
    hb = _update_path(ab_ref[...], *pb)
    h = (jnp.dot(hu, wc1, preferred_element_type=jnp.float32)
         + jnp.dot(hd, wc2, preferred_element_type=jnp.float32)
         + jnp.dot(hb, wc3, preferred_element_type=jnp.float32)) + bc
    o_ref[...] = _bn_relu(h, gc, bec)


def _final(agg_u, agg_d, agg_b, pu, pd, pb, wc, bc, gc, bec):
    def flat(p):
        w1, b1, g1, be1, w2, b2, g2, be2 = p
        return [w1, b1.reshape(1, H), g1.reshape(1, H), be1.reshape(1, H),
                w2, b2.reshape(1, H), g2.reshape(1, H), be2.reshape(1, H)]

    args = ([agg_u, agg_d, agg_b] + flat(pu) + flat(pd) + flat(pb)
            + [wc[0:H], wc[H:2 * H], wc[2 * H:3 * H],
               bc.reshape(1, H), gc.reshape(1, H), bec.reshape(1, H)])
    return pl.pallas_call(
        _final_body,
        out_shape=jax.ShapeDtypeStruct((N, H), jnp.float32),
    )(*args)


@jax.jit
def kernel(x, up_index, up_attr, down_index, down_attr, boundary_index, params):
    wmu, bmu, wmd, bmd, pu, pd, pb, wc, bc, gc, bec = params
    yu, yd = _prep(x, wmu[:D], bmu, wmd[:D], bmd)
    tu = _edge_matmul(up_attr, wmu[D:])
    td = _edge_matmul(down_attr, wmd[D:])
    agg_u, agg_d, agg_b = _sc_aggregate(
        yu, tu, up_index[0], up_index[1],
        yd, td, down_index[0], down_index[1],
        x, boundary_index[0], boundary_index[1])
    return _final(agg_u, agg_d, agg_b, pu, pd, pb, wc, bc, gc, bec)


# speedup vs baseline: 1.5847x; 1.5847x over previous
import functools

import jax
import jax.numpy as jnp
from jax import lax
from jax.experimental import pallas as pl
from jax.experimental.pallas import tpu as pltpu
from jax.experimental.pallas import tpu_sc as plsc

N = 10000
D = 128
H = 128
E_UP = 320000
E_DOWN = 160000
E_B = 160000

NC = 2
NS = 16
L = 16
C = 40
RPT = 624
RTAIL = N - RPT * NS


def _prep_body(x_ref, au_ref, bu_ref, ad_ref, bd_ref, yu_ref, yd_ref):
    x = x_ref[...]
    yu_ref[...] = jnp.dot(x, au_ref[...], preferred_element_type=jnp.float32) + bu_ref[...]
    yd_ref[...] = jnp.dot(x, ad_ref[...], preferred_element_type=jnp.float32) + bd_ref[...]


def _prep(x, au, bu, ad, bd):
    return pl.pallas_call(
        _prep_body,
        out_shape=[jax.ShapeDtypeStruct((N, H), jnp.float32)] * 2,
    )(x, au, bu.reshape(1, H), ad, bd.reshape(1, H))


def _t_body(attr_ref, w_ref, t_ref):
    t_ref[...] = jnp.dot(attr_ref[...], w_ref[...], preferred_element_type=jnp.float32)


def _edge_matmul(attr, w, block=1280):
    e = attr.shape[0]
    return pl.pallas_call(
        _t_body,
        grid=(e // block,),
        in_specs=[
            pl.BlockSpec((block, D), lambda i: (i, 0)),
            pl.BlockSpec((D, H), lambda i: (0, 0)),
        ],
        out_specs=pl.BlockSpec((block, H), lambda i: (i, 0)),
        out_shape=jax.ShapeDtypeStruct((e, H), jnp.float32),
    )(attr, w)


def _sc_body(yu, tu, u0, u1, yd, td, d0, d1, x, b0, b1,
             out_up, out_dn, out_b,
             acc, tb, gb, i0b, i1b, in_sem, g_sem, sc_sem, i1_sem):
    c = lax.axis_index("c")
    s = lax.axis_index("s")

    def rows_copy(src, dst):
        r = pl.ds(pl.multiple_of(s * RPT, 8), RPT)
        pltpu.sync_copy(src.at[r], dst.at[r])

        @pl.when(s == 0)
        def _():
            rt = pl.ds(RPT * NS, RTAIL)
            pltpu.sync_copy(src.at[rt], dst.at[rt])

    def init_acc():
        rows_copy(x, acc)

    def edge_loop(y_ref, t_ref, i0_ref, i1_ref, edges_per_tile, with_t):
        n = edges_per_tile // C
        tile_base = s * edges_per_tile

        def issue_in(jj, s4):
            b = pl.multiple_of(tile_base + jj * C, 8)
            pltpu.async_copy(i0_ref.at[pl.ds(b, C)], i0b.at[s4], in_sem.at[s4])
            if with_t:
                pltpu.async_copy(t_ref.at[pl.ds(b, C)], tb.at[s4], in_sem.at[s4])

        def wait_in(s4):
            pltpu.make_async_copy(i0_ref.at[pl.ds(0, C)], i0b.at[s4], in_sem.at[s4]).wait()
            if with_t:
                pltpu.make_async_copy(t_ref.at[pl.ds(0, C)], tb.at[s4], in_sem.at[s4]).wait()

        def chunk(j, s4, wait_sc, pre_in):
            if wait_sc:
                pltpu.make_async_copy(gb.at[s4], acc.at[i1b.at[s4]], sc_sem.at[s4]).wait()
            b = pl.multiple_of(tile_base + j * C, 8)
            pltpu.async_copy(i1_ref.at[pl.ds(b, C)], i1b.at[s4], i1_sem.at[s4])
            wait_in(s4)
            pltpu.async_copy(y_ref.at[i0b.at[s4]], gb.at[s4], g_sem.at[s4])
            if pre_in:
                issue_in(j + 2, (s4 + 2) % 4)
            pltpu.make_async_copy(y_ref.at[i0b.at[s4]], gb.at[s4], g_sem.at[s4]).wait()
            if with_t:
                gs = gb.at[s4]
                ts = tb.at[s4]

                @plsc.parallel_loop(0, C, 1, unroll=4)
                def _(r):
                    for cb in range(H // L):
                        sl = pl.ds(cb * L, L)
                        gs[r, sl] = jnp.maximum(gs[r, sl] + ts[r, sl], 0.0)
            pltpu.make_async_copy(i1_ref.at[pl.ds(0, C)], i1b.at[s4], i1_sem.at[s4]).wait()
            pltpu.async_copy(gb.at[s4], acc.at[i1b.at[s4]], sc_sem.at[s4], add=True)

        issue_in(0, 0)
        issue_in(1, 1)
        for j in range(4):
            chunk(j, j % 4, False, j + 2 < n)
        g4 = n // 4

        def group(jg, carry):
            for r in range(4):
                chunk(jg * 4 + r, r, True, True)
            return carry

        lax.fori_loop(1, g4 - 1, group, 0, unroll=False)
        for j in range(4 * (g4 - 1), n):
            chunk(j, j % 4, True, j + 2 < n)
        for j in range(n - 4, n):
            s4 = j % 4
            pltpu.make_async_copy(gb.at[s4], acc.at[i1b.at[s4]], sc_sem.at[s4]).wait()

    init_acc()
    plsc.subcore_barrier()

    @pl.when(c == 0)
    def _():
        edge_loop(yu, tu, u0, u1, E_UP // NS, True)

    @pl.when(c == 1)
    def _():
        edge_loop(yd, td, d0, d1, E_DOWN // NS, True)

    plsc.subcore_barrier()

    @pl.when(c == 0)
    def _():
        rows_copy(acc, out_up)

    @pl.when(c == 1)
    def _():
        rows_copy(acc, out_dn)

    plsc.subcore_barrier()

    @pl.when(c == 1)
    def _():
        init_acc()

    plsc.subcore_barrier()

    @pl.when(c == 1)
    def _():
        edge_loop(x, None, b0, b1, E_B // NS, False)

    plsc.subcore_barrier()

    @pl.when(c == 1)
    def _():
        rows_copy(acc, out_b)


def _sc_aggregate(yu, tu, u0, u1, yd, td, d0, d1, x, b0, b1):
    mesh = plsc.VectorSubcoreMesh(
        core_axis_name="c", subcore_axis_name="s", num_cores=NC, num_subcores=NS)
    return pl.kernel(
        _sc_body,
        out_type=[jax.ShapeDtypeStruct((N, H), jnp.float32)] * 3,
        mesh=mesh,
        scratch_types=[
            pltpu.VMEM_SHARED((N, H), jnp.float32),
            pltpu.VMEM((4, C, H), jnp.float32),
            pltpu.VMEM((4, C, H), jnp.float32),
            pltpu.VMEM((4, C), jnp.int32),
            pltpu.VMEM((4, C), jnp.int32),
            pltpu.SemaphoreType.DMA((4,)),
            pltpu.SemaphoreType.DMA((4,)),
            pltpu.SemaphoreType.DMA((4,)),
            pltpu.SemaphoreType.DMA((4,)),
        ],
    )(yu, tu, u0, u1, yd, td, d0, d1, x, b0, b1)


def _bn_relu(h, g, b):
    mu = jnp.mean(h, axis=0, keepdims=True)
    var = jnp.mean((h - mu) * (h - mu), axis=0, keepdims=True)
    return jnp.maximum((h - mu) * lax.rsqrt(var + 1e-5) * g + b, 0.0)


def _update_path(h, w1, b1, g1, be1, w2, b2, g2, be2):
    h = _bn_relu(jnp.dot(h, w1, preferred_element_type=jnp.float32) + b1, g1, be1)
    h = _bn_relu(jnp.dot(h, w2, preferred_element_type=jnp.float32) + b2, g2, be2)
    return h


def _final_body(au_ref, ad_ref, ab_ref, *refs):
    prefs = refs[:-1]
    o_ref = refs[-1]
    pu = [r[...] for r in prefs[0:8]]
    pd = [r[...] for r in prefs[8:16]]
    pb = [r[...] for r in prefs[16:24]]
    wc1, wc2, wc3, bc, gc, bec = [r[...] for r in prefs[24:30]]
    hu = _update_path(au_ref[...], *pu)
    hd = _update_path(ad_ref[...], *pd)
    hb = _update_path(ab_ref[...], *pb)
    h = (jnp.dot(hu, wc1, preferred_element_type=jnp.float32)
         + jnp.dot(hd, wc2, preferred_element_type=jnp.float32)
         + jnp.dot(hb, wc3, preferred_element_type=jnp.float32)) + bc
    o_ref[...] = _bn_relu(h, gc, bec)


def _final(agg_u, agg_d, agg_b, pu, pd, pb, wc, bc, gc, bec):
    def flat(p):
        w1, b1, g1, be1, w2, b2, g2, be2 = p
        return [w1, b1.reshape(1, H), g1.reshape(1, H), be1.reshape(1, H),
                w2, b2.reshape(1, H), g2.reshape(1, H), be2.reshape(1, H)]

    args = ([agg_u, agg_d, agg_b] + flat(pu) + flat(pd) + flat(pb)
            + [wc[0:H], wc[H:2 * H], wc[2 * H:3 * H],
               bc.reshape(1, H), gc.reshape(1, H), bec.reshape(1, H)])
    return pl.pallas_call(
        _final_body,
        out_shape=jax.ShapeDtypeStruct((N, H), jnp.float32),
    )(*args)


@jax.jit
def kernel(x, up_index, up_attr, down_index, down_attr, boundary_index, params):
    wmu, bmu, wmd, bmd, pu, pd, pb, wc, bc, gc, bec = params
    yu, yd = _prep(x, wmu[:D], bmu, wmd[:D], bmd)
    tu = _edge_matmul(up_attr, wmu[D:])
    td = _edge_matmul(down_attr, wmd[D:])
    agg_u, agg_d, agg_b = _sc_aggregate(
        yu, tu, up_index[0], up_index[1],
        yd, td, down_index[0], down_index[1],
        x, boundary_index[0], boundary_index[1])
    return _final(agg_u, agg_d, agg_b, pu, pd, pb, wc, bc, gc, bec)


# speedup vs baseline: 1.9210x; 1.2122x over previous
import functools

import jax
import jax.numpy as jnp
from jax import lax
from jax.experimental import pallas as pl
from jax.experimental.pallas import tpu as pltpu
from jax.experimental.pallas import tpu_sc as plsc

N = 10000
D = 128
H = 128
E_UP = 320000
E_DOWN = 160000
E_B = 160000

NC = 2
NS = 16
L = 16
C = 40
RPT = 624
RTAIL = N - RPT * NS


def _prep_body(x_ref, au_ref, bu_ref, ad_ref, bd_ref, yu_ref, yd_ref):
    x = x_ref[...]
    yu_ref[...] = jnp.dot(x, au_ref[...], preferred_element_type=jnp.float32) + bu_ref[...]
    yd_ref[...] = jnp.dot(x, ad_ref[...], preferred_element_type=jnp.float32) + bd_ref[...]


def _prep(x, au, bu, ad, bd):
    return pl.pallas_call(
        _prep_body,
        out_shape=[jax.ShapeDtypeStruct((N, H), jnp.float32)] * 2,
    )(x, au, bu.reshape(1, H), ad, bd.reshape(1, H))


def _t_body(attr_ref, w_ref, t_ref):
    t_ref[...] = jnp.dot(attr_ref[...], w_ref[...], preferred_element_type=jnp.float32)


def _edge_matmul(attr, w, block=1280):
    e = attr.shape[0]
    return pl.pallas_call(
        _t_body,
        grid=(e // block,),
        in_specs=[
            pl.BlockSpec((block, D), lambda i: (i, 0)),
            pl.BlockSpec((D, H), lambda i: (0, 0)),
        ],
        out_specs=pl.BlockSpec((block, H), lambda i: (i, 0)),
        out_shape=jax.ShapeDtypeStruct((e, H), jnp.float32),
    )(attr, w)


def _sc_body(yu, tu, u0, u1, yd, td, d0, d1, x, b0, b1,
             out_up, out_dn, out_b,
             acc, tb, gb, i0b, i1b, in_sem, g_sem, sc_sem, i1_sem):
    c = lax.axis_index("c")
    s = lax.axis_index("s")

    def rows_copy(src, dst):
        r = pl.ds(pl.multiple_of(s * RPT, 8), RPT)
        pltpu.sync_copy(src.at[r], dst.at[r])

        @pl.when(s == 0)
        def _():
            rt = pl.ds(RPT * NS, RTAIL)
            pltpu.sync_copy(src.at[rt], dst.at[rt])

    def init_acc():
        rows_copy(x, acc)

    def edge_loop(y_ref, t_ref, i0_ref, i1_ref, edges_per_tile, with_t):
        n = edges_per_tile // C
        tile_base = s * edges_per_tile

        def issue_in(jj, s4):
            b = pl.multiple_of(tile_base + jj * C, 8)
            pltpu.async_copy(i0_ref.at[pl.ds(b, C)], i0b.at[s4], in_sem.at[s4])
            if with_t:
                pltpu.async_copy(t_ref.at[pl.ds(b, C)], tb.at[s4], in_sem.at[s4])

        def wait_in(s4):
            pltpu.make_async_copy(i0_ref.at[pl.ds(0, C)], i0b.at[s4], in_sem.at[s4]).wait()
            if with_t:
                pltpu.make_async_copy(t_ref.at[pl.ds(0, C)], tb.at[s4], in_sem.at[s4]).wait()

        def issue_gather_i1(jj, s4):
            wait_in(s4)
            pltpu.async_copy(y_ref.at[i0b.at[s4]], gb.at[s4], g_sem.at[s4])
            b = pl.multiple_of(tile_base + jj * C, 8)
            pltpu.async_copy(i1_ref.at[pl.ds(b, C)], i1b.at[s4], i1_sem.at[s4])

        def chunk(j, s4, wait_sc, pre_g, pre_in):
            s4p = (s4 + 1) % 4
            if wait_sc:
                pltpu.make_async_copy(gb.at[s4p], acc.at[i1b.at[s4p]], sc_sem.at[s4p]).wait()
            if pre_g:
                issue_gather_i1(j + 1, s4p)
            if pre_in:
                issue_in(j + 2, (s4 + 2) % 4)
            pltpu.make_async_copy(y_ref.at[i0b.at[s4]], gb.at[s4], g_sem.at[s4]).wait()
            if with_t:
                gs = gb.at[s4]
                ts = tb.at[s4]

                @plsc.parallel_loop(0, C, 1, unroll=4)
                def _(r):
                    for cb in range(H // L):
                        sl = pl.ds(cb * L, L)
                        gs[r, sl] = jnp.maximum(gs[r, sl] + ts[r, sl], 0.0)
            pltpu.make_async_copy(i1_ref.at[pl.ds(0, C)], i1b.at[s4], i1_sem.at[s4]).wait()
            pltpu.async_copy(gb.at[s4], acc.at[i1b.at[s4]], sc_sem.at[s4], add=True)

        issue_in(0, 0)
        issue_in(1, 1)
        issue_gather_i1(0, 0)
        for j in range(4):
            chunk(j, j % 4, j >= 3, j + 1 < n, j + 2 < n)
        g4 = n // 4

        def group(jg, carry):
            for r in range(4):
                chunk(jg * 4 + r, r, True, True, True)
            return carry

        lax.fori_loop(1, g4 - 1, group, 0, unroll=False)
        for j in range(4 * (g4 - 1), n):
            chunk(j, j % 4, j >= 3, j + 1 < n, j + 2 < n)
        for j in range(n - 3, n):
            s4 = j % 4
            pltpu.make_async_copy(gb.at[s4], acc.at[i1b.at[s4]], sc_sem.at[s4]).wait()

    init_acc()
    plsc.subcore_barrier()

    @pl.when(c == 0)
    def _():
        edge_loop(yu, tu, u0, u1, E_UP // NS, True)

    @pl.when(c == 1)
    def _():
        edge_loop(yd, td, d0, d1, E_DOWN // NS, True)

    plsc.subcore_barrier()

    @pl.when(c == 0)
    def _():
        rows_copy(acc, out_up)

    @pl.when(c == 1)
    def _():
        rows_copy(acc, out_dn)

    plsc.subcore_barrier()

    @pl.when(c == 1)
    def _():
        init_acc()

    plsc.subcore_barrier()

    @pl.when(c == 1)
    def _():
        edge_loop(x, None, b0, b1, E_B // NS, False)

    plsc.subcore_barrier()

    @pl.when(c == 1)
    def _():
        rows_copy(acc, out_b)


def _sc_aggregate(yu, tu, u0, u1, yd, td, d0, d1, x, b0, b1):
    mesh = plsc.VectorSubcoreMesh(
        core_axis_name="c", subcore_axis_name="s", num_cores=NC, num_subcores=NS)
    return pl.kernel(
        _sc_body,
        out_type=[jax.ShapeDtypeStruct((N, H), jnp.float32)] * 3,
        mesh=mesh,
        scratch_types=[
            pltpu.VMEM_SHARED((N, H), jnp.float32),
            pltpu.VMEM((4, C, H), jnp.float32),
            pltpu.VMEM((4, C, H), jnp.float32),
            pltpu.VMEM((4, C), jnp.int32),
            pltpu.VMEM((4, C), jnp.int32),
            pltpu.SemaphoreType.DMA((4,)),
            pltpu.SemaphoreType.DMA((4,)),
            pltpu.SemaphoreType.DMA((4,)),
            pltpu.SemaphoreType.DMA((4,)),
        ],
    )(yu, tu, u0, u1, yd, td, d0, d1, x, b0, b1)


def _bn_relu(h, g, b):
    mu = jnp.mean(h, axis=0, keepdims=True)
    var = jnp.mean((h - mu) * (h - mu), axis=0, keepdims=True)
    return jnp.maximum((h - mu) * lax.rsqrt(var + 1e-5) * g + b, 0.0)


def _update_path(h, w1, b1, g1, be1, w2, b2, g2, be2):
    h = _bn_relu(jnp.dot(h, w1, preferred_element_type=jnp.float32) + b1, g1, be1)
    h = _bn_relu(jnp.dot(h, w2, preferred_element_type=jnp.float32) + b2, g2, be2)
    return h


def _final_body(au_ref, ad_ref, ab_ref, *refs):
    prefs = refs[:-1]
    o_ref = refs[-1]
    pu = [r[...] for r in prefs[0:8]]
    pd = [r[...] for r in prefs[8:16]]
    pb = [r[...] for r in prefs[16:24]]
    wc1, wc2, wc3, bc, gc, bec = [r[...] for r in prefs[24:30]]
    hu = _update_path(au_ref[...], *pu)
    hd = _update_path(ad_ref[...], *pd)
    hb = _update_path(ab_ref[...], *pb)
    h = (jnp.dot(hu, wc1, preferred_element_type=jnp.float32)
         + jnp.dot(hd, wc2, preferred_element_type=jnp.float32)
         + jnp.dot(hb, wc3, preferred_element_type=jnp.float32)) + bc
    o_ref[...] = _bn_relu(h, gc, bec)


def _final(agg_u, agg_d, agg_b, pu, pd, pb, wc, bc, gc, bec):
    def flat(p):
        w1, b1, g1, be1, w2, b2, g2, be2 = p
        return [w1, b1.reshape(1, H), g1.reshape(1, H), be1.reshape(1, H),
                w2, b2.reshape(1, H), g2.reshape(1, H), be2.reshape(1, H)]

    args = ([agg_u, agg_d, agg_b] + flat(pu) + flat(pd) + flat(pb)
            + [wc[0:H], wc[H:2 * H], wc[2 * H:3 * H],
               bc.reshape(1, H), gc.reshape(1, H), bec.reshape(1, H)])
    return pl.pallas_call(
        _final_body,
        out_shape=jax.ShapeDtypeStruct((N, H), jnp.float32),
    )(*args)


@jax.jit
def kernel(x, up_index, up_attr, down_index, down_attr, boundary_index, params):
    wmu, bmu, wmd, bmd, pu, pd, pb, wc, bc, gc, bec = params
    yu, yd = _prep(x, wmu[:D], bmu, wmd[:D], bmd)
    tu = _edge_matmul(up_attr, wmu[D:])
    td = _edge_matmul(down_attr, wmd[D:])
    agg_u, agg_d, agg_b = _sc_aggregate(
        yu, tu, up_index[0], up_index[1],
        yd, td, down_index[0], down_index[1],
        x, boundary_index[0], boundary_index[1])
    return _final(agg_u, agg_d, agg_b, pu, pd, pb, wc, bc, gc, bec)


# speedup vs baseline: 2.2740x; 1.1838x over previous
import functools

import jax
import jax.numpy as jnp
from jax import lax
from jax.experimental import pallas as pl
from jax.experimental.pallas import tpu as pltpu
from jax.experimental.pallas import tpu_sc as plsc

N = 10000
D = 128
H = 128
E_UP = 320000
E_DOWN = 160000
E_B = 160000

NC = 2
NS = 16
L = 16
C = 40
RPT = 624
RTAIL = N - RPT * NS


BU = E_UP // 125
BD = E_DOWN // 125


def _front_body(x_ref, au_ref, bu_ref, ad_ref, bd_ref, ua_ref, wu_ref,
                da_ref, wd_ref, yu_ref, yd_ref, tu_ref, td_ref):
    @pl.when(pl.program_id(0) == 0)
    def _():
        xx = x_ref[...]
        yu_ref[...] = jnp.dot(xx, au_ref[...], preferred_element_type=jnp.float32) + bu_ref[...]
        yd_ref[...] = jnp.dot(xx, ad_ref[...], preferred_element_type=jnp.float32) + bd_ref[...]

    tu_ref[...] = jnp.dot(ua_ref[...], wu_ref[...], preferred_element_type=jnp.float32)
    td_ref[...] = jnp.dot(da_ref[...], wd_ref[...], preferred_element_type=jnp.float32)


def _front(x, au, bu, ad, bd, up_attr, wu, down_attr, wd):
    zero = lambda i: (0, 0)
    return pl.pallas_call(
        _front_body,
        grid=(125,),
        in_specs=[
            pl.BlockSpec((N, D), zero),
            pl.BlockSpec((D, H), zero),
            pl.BlockSpec((1, H), zero),
            pl.BlockSpec((D, H), zero),
            pl.BlockSpec((1, H), zero),
            pl.BlockSpec((BU, D), lambda i: (i, 0)),
            pl.BlockSpec((D, H), zero),
            pl.BlockSpec((BD, D), lambda i: (i, 0)),
            pl.BlockSpec((D, H), zero),
        ],
        out_specs=[
            pl.BlockSpec((N, H), zero),
            pl.BlockSpec((N, H), zero),
            pl.BlockSpec((BU, H), lambda i: (i, 0)),
            pl.BlockSpec((BD, H), lambda i: (i, 0)),
        ],
        out_shape=[
            jax.ShapeDtypeStruct((N, H), jnp.float32),
            jax.ShapeDtypeStruct((N, H), jnp.float32),
            jax.ShapeDtypeStruct((E_UP, H), jnp.float32),
            jax.ShapeDtypeStruct((E_DOWN, H), jnp.float32),
        ],
    )(x, au, bu.reshape(1, H), ad, bd.reshape(1, H), up_attr, wu, down_attr, wd)


def _sc_body(yu, tu, u0, u1, yd, td, d0, d1, x, b0, b1,
             out_up, out_dn, out_b,
             acc, tb, gb, i0b, i1b, in_sem, g_sem, sc_sem, i1_sem):
    c = lax.axis_index("c")
    s = lax.axis_index("s")

    def rows_copy(src, dst):
        r = pl.ds(pl.multiple_of(s * RPT, 8), RPT)
        pltpu.sync_copy(src.at[r], dst.at[r])

        @pl.when(s == 0)
        def _():
            rt = pl.ds(RPT * NS, RTAIL)
            pltpu.sync_copy(src.at[rt], dst.at[rt])

    def init_acc():
        rows_copy(x, acc)

    def edge_loop(y_ref, t_ref, i0_ref, i1_ref, edges_per_tile, with_t):
        n = edges_per_tile // C
        tile_base = s * edges_per_tile

        def issue_in(jj, s4):
            b = pl.multiple_of(tile_base + jj * C, 8)
            pltpu.async_copy(i0_ref.at[pl.ds(b, C)], i0b.at[s4], in_sem.at[s4])
            if with_t:
                pltpu.async_copy(t_ref.at[pl.ds(b, C)], tb.at[s4], in_sem.at[s4])

        def wait_in(s4):
            pltpu.make_async_copy(i0_ref.at[pl.ds(0, C)], i0b.at[s4], in_sem.at[s4]).wait()
            if with_t:
                pltpu.make_async_copy(t_ref.at[pl.ds(0, C)], tb.at[s4], in_sem.at[s4]).wait()

        def issue_gather_i1(jj, s4):
            wait_in(s4)
            pltpu.async_copy(y_ref.at[i0b.at[s4]], gb.at[s4], g_sem.at[s4])
            b = pl.multiple_of(tile_base + jj * C, 8)
            pltpu.async_copy(i1_ref.at[pl.ds(b, C)], i1b.at[s4], i1_sem.at[s4])

        def chunk(j, s4, wait_sc, pre_g, pre_in):
            s4p = (s4 + 1) % 4
            if wait_sc:
                pltpu.make_async_copy(gb.at[s4p], acc.at[i1b.at[s4p]], sc_sem.at[s4p]).wait()
            if pre_g:
                issue_gather_i1(j + 1, s4p)
            if pre_in:
                issue_in(j + 2, (s4 + 2) % 4)
            pltpu.make_async_copy(y_ref.at[i0b.at[s4]], gb.at[s4], g_sem.at[s4]).wait()
            if with_t:
                gs = gb.at[s4]
                ts = tb.at[s4]

                @plsc.parallel_loop(0, C, 1, unroll=8)
                def _(r):
                    for cb in range(H // L):
                        sl = pl.ds(cb * L, L)
                        gs[r, sl] = jnp.maximum(gs[r, sl] + ts[r, sl], 0.0)
            pltpu.make_async_copy(i1_ref.at[pl.ds(0, C)], i1b.at[s4], i1_sem.at[s4]).wait()
            pltpu.async_copy(gb.at[s4], acc.at[i1b.at[s4]], sc_sem.at[s4], add=True)

        issue_in(0, 0)
        issue_in(1, 1)
        issue_gather_i1(0, 0)
        for j in range(4):
            chunk(j, j % 4, j >= 3, j + 1 < n, j + 2 < n)
        g4 = n // 4

        def group(jg, carry):
            for r in range(4):
                chunk(jg * 4 + r, r, True, True, True)
            return carry

        lax.fori_loop(1, g4 - 1, group, 0, unroll=False)
        for j in range(4 * (g4 - 1), n):
            chunk(j, j % 4, j >= 3, j + 1 < n, j + 2 < n)
        for j in range(n - 3, n):
            s4 = j % 4
            pltpu.make_async_copy(gb.at[s4], acc.at[i1b.at[s4]], sc_sem.at[s4]).wait()

    init_acc()
    plsc.subcore_barrier()

    @pl.when(c == 0)
    def _():
        edge_loop(yu, tu, u0, u1, E_UP // NS, True)

    @pl.when(c == 1)
    def _():
        edge_loop(yd, td, d0, d1, E_DOWN // NS, True)

    plsc.subcore_barrier()

    @pl.when(c == 0)
    def _():
        rows_copy(acc, out_up)

    @pl.when(c == 1)
    def _():
        rows_copy(acc, out_dn)

    plsc.subcore_barrier()

    @pl.when(c == 1)
    def _():
        init_acc()

    plsc.subcore_barrier()

    @pl.when(c == 1)
    def _():
        edge_loop(x, None, b0, b1, E_B // NS, False)

    plsc.subcore_barrier()

    @pl.when(c == 1)
    def _():
        rows_copy(acc, out_b)


def _sc_aggregate(yu, tu, u0, u1, yd, td, d0, d1, x, b0, b1):
    mesh = plsc.VectorSubcoreMesh(
        core_axis_name="c", subcore_axis_name="s", num_cores=NC, num_subcores=NS)
    return pl.kernel(
        _sc_body,
        out_type=[jax.ShapeDtypeStruct((N, H), jnp.float32)] * 3,
        mesh=mesh,
        scratch_types=[
            pltpu.VMEM_SHARED((N, H), jnp.float32),
            pltpu.VMEM((4, C, H), jnp.float32),
            pltpu.VMEM((4, C, H), jnp.float32),
            pltpu.VMEM((4, C), jnp.int32),
            pltpu.VMEM((4, C), jnp.int32),
            pltpu.SemaphoreType.DMA((4,)),
            pltpu.SemaphoreType.DMA((4,)),
            pltpu.SemaphoreType.DMA((4,)),
            pltpu.SemaphoreType.DMA((4,)),
        ],
    )(yu, tu, u0, u1, yd, td, d0, d1, x, b0, b1)


def _bn_relu(h, g, b):
    mu = jnp.mean(h, axis=0, keepdims=True)
    var = jnp.mean((h - mu) * (h - mu), axis=0, keepdims=True)
    return jnp.maximum((h - mu) * lax.rsqrt(var + 1e-5) * g + b, 0.0)


def _update_path(h, w1, b1, g1, be1, w2, b2, g2, be2):
    h = _bn_relu(jnp.dot(h, w1, preferred_element_type=jnp.float32) + b1, g1, be1)
    h = _bn_relu(jnp.dot(h, w2, preferred_element_type=jnp.float32) + b2, g2, be2)
    return h


def _final_body(au_ref, ad_ref, ab_ref, *refs):
    prefs = refs[:-1]
    o_ref = refs[-1]
    pu = [r[...] for r in prefs[0:8]]
    pd = [r[...] for r in prefs[8:16]]
    pb = [r[...] for r in prefs[16:24]]
    wc1, wc2, wc3, bc, gc, bec = [r[...] for r in prefs[24:30]]
    hu = _update_path(au_ref[...], *pu)
    hd = _update_path(ad_ref[...], *pd)
    hb = _update_path(ab_ref[...], *pb)
    h = (jnp.dot(hu, wc1, preferred_element_type=jnp.float32)
         + jnp.dot(hd, wc2, preferred_element_type=jnp.float32)
         + jnp.dot(hb, wc3, preferred_element_type=jnp.float32)) + bc
    o_ref[...] = _bn_relu(h, gc, bec)


def _final(agg_u, agg_d, agg_b, pu, pd, pb, wc, bc, gc, bec):
    def flat(p):
        w1, b1, g1, be1, w2, b2, g2, be2 = p
        return [w1, b1.reshape(1, H), g1.reshape(1, H), be1.reshape(1, H),
                w2, b2.reshape(1, H), g2.reshape(1, H), be2.reshape(1, H)]

    args = ([agg_u, agg_d, agg_b] + flat(pu) + flat(pd) + flat(pb)
            + [wc[0:H], wc[H:2 * H], wc[2 * H:3 * H],
               bc.reshape(1, H), gc.reshape(1, H), bec.reshape(1, H)])
    return pl.pallas_call(
        _final_body,
        out_shape=jax.ShapeDtypeStruct((N, H), jnp.float32),
    )(*args)


@jax.jit
def kernel(x, up_index, up_attr, down_index, down_attr, boundary_index, params):
    wmu, bmu, wmd, bmd, pu, pd, pb, wc, bc, gc, bec = params
    yu, yd, tu, td = _front(x, wmu[:D], bmu, wmd[:D], bmd,
                            up_attr, wmu[D:], down_attr, wmd[D:])
    agg_u, agg_d, agg_b = _sc_aggregate(
        yu, tu, up_index[0], up_index[1],
        yd, td, down_index[0], down_index[1],
        x, boundary_index[0], boundary_index[1])
    return _final(agg_u, agg_d, agg_b, pu, pd, pb, wc, bc, gc, bec)


# speedup vs baseline: 2.2822x; 1.0036x over previous
import functools

import jax
import jax.numpy as jnp
from jax import lax
from jax.experimental import pallas as pl
from jax.experimental.pallas import tpu as pltpu
from jax.experimental.pallas import tpu_sc as plsc

N = 10000
D = 128
H = 128
E_UP = 320000
E_DOWN = 160000
E_B = 160000

NC = 2
NS = 16
L = 16
C = 40
RPT = 624
RTAIL = N - RPT * NS


BU = E_UP // 125
BD = E_DOWN // 125


def _front_body(x_ref, au_ref, bu_ref, ad_ref, bd_ref, ua_ref, wu_ref,
                da_ref, wd_ref, yu_ref, yd_ref, tu_ref, td_ref):
    @pl.when(pl.program_id(0) == 0)
    def _():
        xx = x_ref[...]
        yu_ref[...] = jnp.dot(xx, au_ref[...], preferred_element_type=jnp.float32) + bu_ref[...]
        yd_ref[...] = jnp.dot(xx, ad_ref[...], preferred_element_type=jnp.float32) + bd_ref[...]

    tu_ref[...] = jnp.dot(ua_ref[...], wu_ref[...], preferred_element_type=jnp.float32)
    td_ref[...] = jnp.dot(da_ref[...], wd_ref[...], preferred_element_type=jnp.float32)


def _front(x, au, bu, ad, bd, up_attr, wu, down_attr, wd):
    zero = lambda i: (0, 0)
    return pl.pallas_call(
        _front_body,
        grid=(125,),
        in_specs=[
            pl.BlockSpec((N, D), zero),
            pl.BlockSpec((D, H), zero),
            pl.BlockSpec((1, H), zero),
            pl.BlockSpec((D, H), zero),
            pl.BlockSpec((1, H), zero),
            pl.BlockSpec((BU, D), lambda i: (i, 0)),
            pl.BlockSpec((D, H), zero),
            pl.BlockSpec((BD, D), lambda i: (i, 0)),
            pl.BlockSpec((D, H), zero),
        ],
        out_specs=[
            pl.BlockSpec((N, H), zero),
            pl.BlockSpec((N, H), zero),
            pl.BlockSpec((BU, H), lambda i: (i, 0)),
            pl.BlockSpec((BD, H), lambda i: (i, 0)),
        ],
        out_shape=[
            jax.ShapeDtypeStruct((N, H), jnp.float32),
            jax.ShapeDtypeStruct((N, H), jnp.float32),
            jax.ShapeDtypeStruct((E_UP, H), jnp.float32),
            jax.ShapeDtypeStruct((E_DOWN, H), jnp.float32),
        ],
    )(x, au, bu.reshape(1, H), ad, bd.reshape(1, H), up_attr, wu, down_attr, wd)


def _sc_body(yu, tu, u0, u1, yd, td, d0, d1, x, b0, b1,
             out_up, out_dn, out_b,
             acc, tb, gb, i0b, i1b, in_sem, g_sem, sc_sem, i1_sem):
    c = lax.axis_index("c")
    s = lax.axis_index("s")

    def rows_copy(src, dst):
        r = pl.ds(pl.multiple_of(s * RPT, 8), RPT)
        pltpu.sync_copy(src.at[r], dst.at[r])

        @pl.when(s == 0)
        def _():
            rt = pl.ds(RPT * NS, RTAIL)
            pltpu.sync_copy(src.at[rt], dst.at[rt])

    def init_acc():
        rows_copy(x, acc)

    def edge_loop(y_ref, t_ref, i0_ref, i1_ref, edges_per_tile, with_t):
        n = edges_per_tile // C
        tile_base = s * edges_per_tile

        def issue_in(jj, s4):
            b = pl.multiple_of(tile_base + jj * C, 8)
            pltpu.async_copy(i0_ref.at[pl.ds(b, C)], i0b.at[s4], in_sem.at[s4])
            if with_t:
                pltpu.async_copy(t_ref.at[pl.ds(b, C)], tb.at[s4], in_sem.at[s4])

        def wait_in(s4):
            pltpu.make_async_copy(i0_ref.at[pl.ds(0, C)], i0b.at[s4], in_sem.at[s4]).wait()
            if with_t:
                pltpu.make_async_copy(t_ref.at[pl.ds(0, C)], tb.at[s4], in_sem.at[s4]).wait()

        def issue_gather_i1(jj, s4):
            wait_in(s4)
            pltpu.async_copy(y_ref.at[i0b.at[s4]], gb.at[s4], g_sem.at[s4])
            b = pl.multiple_of(tile_base + jj * C, 8)
            pltpu.async_copy(i1_ref.at[pl.ds(b, C)], i1b.at[s4], i1_sem.at[s4])

        def chunk(j, s4, wait_sc, pre_g, pre_in):
            s4pp = (s4 + 2) % 4
            if wait_sc:
                pltpu.make_async_copy(gb.at[s4pp], acc.at[i1b.at[s4pp]], sc_sem.at[s4pp]).wait()
            if pre_g:
                issue_gather_i1(j + 2, s4pp)
            if pre_in:
                issue_in(j + 3, (s4 + 3) % 4)
            pltpu.make_async_copy(y_ref.at[i0b.at[s4]], gb.at[s4], g_sem.at[s4]).wait()
            if with_t:
                gs = gb.at[s4]
                ts = tb.at[s4]

                @plsc.parallel_loop(0, C, 1, unroll=8)
                def _(r):
                    for cb in range(H // L):
                        sl = pl.ds(cb * L, L)
                        gs[r, sl] = jnp.maximum(gs[r, sl] + ts[r, sl], 0.0)
            pltpu.make_async_copy(i1_ref.at[pl.ds(0, C)], i1b.at[s4], i1_sem.at[s4]).wait()
            pltpu.async_copy(gb.at[s4], acc.at[i1b.at[s4]], sc_sem.at[s4], add=True)

        issue_in(0, 0)
        issue_in(1, 1)
        issue_in(2, 2)
        issue_gather_i1(0, 0)
        issue_gather_i1(1, 1)
        for j in range(4):
            chunk(j, j % 4, j >= 2, j + 2 < n, j + 3 < n)
        g4 = n // 4

        def group(jg, carry):
            for r in range(4):
                chunk(jg * 4 + r, r, True, True, True)
            return carry

        lax.fori_loop(1, g4 - 1, group, 0, unroll=False)
        for j in range(4 * (g4 - 1), n):
            chunk(j, j % 4, j >= 2, j + 2 < n, j + 3 < n)
        for j in range(n - 2, n):
            s4 = j % 4
            pltpu.make_async_copy(gb.at[s4], acc.at[i1b.at[s4]], sc_sem.at[s4]).wait()

    init_acc()
    plsc.subcore_barrier()

    @pl.when(c == 0)
    def _():
        edge_loop(yu, tu, u0, u1, E_UP // NS, True)

    @pl.when(c == 1)
    def _():
        edge_loop(yd, td, d0, d1, E_DOWN // NS, True)

    plsc.subcore_barrier()

    @pl.when(c == 0)
    def _():
        rows_copy(acc, out_up)

    @pl.when(c == 1)
    def _():
        rows_copy(acc, out_dn)

    plsc.subcore_barrier()

    @pl.when(c == 1)
    def _():
        init_acc()

    plsc.subcore_barrier()

    @pl.when(c == 1)
    def _():
        edge_loop(x, None, b0, b1, E_B // NS, False)

    plsc.subcore_barrier()

    @pl.when(c == 1)
    def _():
        rows_copy(acc, out_b)


def _sc_aggregate(yu, tu, u0, u1, yd, td, d0, d1, x, b0, b1):
    mesh = plsc.VectorSubcoreMesh(
        core_axis_name="c", subcore_axis_name="s", num_cores=NC, num_subcores=NS)
    return pl.kernel(
        _sc_body,
        out_type=[jax.ShapeDtypeStruct((N, H), jnp.float32)] * 3,
        mesh=mesh,
        scratch_types=[
            pltpu.VMEM_SHARED((N, H), jnp.float32),
            pltpu.VMEM((4, C, H), jnp.float32),
            pltpu.VMEM((4, C, H), jnp.float32),
            pltpu.VMEM((4, C), jnp.int32),
            pltpu.VMEM((4, C), jnp.int32),
            pltpu.SemaphoreType.DMA((4,)),
            pltpu.SemaphoreType.DMA((4,)),
            pltpu.SemaphoreType.DMA((4,)),
            pltpu.SemaphoreType.DMA((4,)),
        ],
    )(yu, tu, u0, u1, yd, td, d0, d1, x, b0, b1)


def _bn_relu(h, g, b):
    mu = jnp.mean(h, axis=0, keepdims=True)
    var = jnp.mean((h - mu) * (h - mu), axis=0, keepdims=True)
    return jnp.maximum((h - mu) * lax.rsqrt(var + 1e-5) * g + b, 0.0)


def _update_path(h, w1, b1, g1, be1, w2, b2, g2, be2):
    h = _bn_relu(jnp.dot(h, w1, preferred_element_type=jnp.float32) + b1, g1, be1)
    h = _bn_relu(jnp.dot(h, w2, preferred_element_type=jnp.float32) + b2, g2, be2)
    return h


def _final_body(au_ref, ad_ref, ab_ref, *refs):
    prefs = refs[:-1]
    o_ref = refs[-1]
    pu = [r[...] for r in prefs[0:8]]
    pd = [r[...] for r in prefs[8:16]]
    pb = [r[...] for r in prefs[16:24]]
    wc1, wc2, wc3, bc, gc, bec = [r[...] for r in prefs[24:30]]
    hu = _update_path(au_ref[...], *pu)
    hd = _update_path(ad_ref[...], *pd)
    hb = _update_path(ab_ref[...], *pb)
    h = (jnp.dot(hu, wc1, preferred_element_type=jnp.float32)
         + jnp.dot(hd, wc2, preferred_element_type=jnp.float32)
         + jnp.dot(hb, wc3, preferred_element_type=jnp.float32)) + bc
    o_ref[...] = _bn_relu(h, gc, bec)


def _final(agg_u, agg_d, agg_b, pu, pd, pb, wc, bc, gc, bec):
    def flat(p):
        w1, b1, g1, be1, w2, b2, g2, be2 = p
        return [w1, b1.reshape(1, H), g1.reshape(1, H), be1.reshape(1, H),
                w2, b2.reshape(1, H), g2.reshape(1, H), be2.reshape(1, H)]

    args = ([agg_u, agg_d, agg_b] + flat(pu) + flat(pd) + flat(pb)
            + [wc[0:H], wc[H:2 * H], wc[2 * H:3 * H],
               bc.reshape(1, H), gc.reshape(1, H), bec.reshape(1, H)])
    return pl.pallas_call(
        _final_body,
        out_shape=jax.ShapeDtypeStruct((N, H), jnp.float32),
    )(*args)


@jax.jit
def kernel(x, up_index, up_attr, down_index, down_attr, boundary_index, params):
    wmu, bmu, wmd, bmd, pu, pd, pb, wc, bc, gc, bec = params
    yu, yd, tu, td = _front(x, wmu[:D], bmu, wmd[:D], bmd,
                            up_attr, wmu[D:], down_attr, wmd[D:])
    agg_u, agg_d, agg_b = _sc_aggregate(
        yu, tu, up_index[0], up_index[1],
        yd, td, down_index[0], down_index[1],
        x, boundary_index[0], boundary_index[1])
    return _final(agg_u, agg_d, agg_b, pu, pd, pb, wc, bc, gc, bec)


# speedup vs baseline: 2.7644x; 1.2113x over previous
import functools

import jax
import jax.numpy as jnp
from jax import lax
from jax.experimental import pallas as pl
from jax.experimental.pallas import tpu as pltpu
from jax.experimental.pallas import tpu_sc as plsc

N = 10000
D = 128
H = 128
E_UP = 320000
E_DOWN = 160000
E_B = 160000

NC = 2
NS = 16
L = 16
C = 40
RPT = 624
RTAIL = N - RPT * NS


BU = E_UP // 125
BD = E_DOWN // 125


def _front_body(x_ref, au_ref, bu_ref, ad_ref, bd_ref, ua_ref, wu_ref,
                da_ref, wd_ref, yu_ref, yd_ref, tu_ref, td_ref):
    @pl.when(pl.program_id(0) == 0)
    def _():
        xx = x_ref[...]
        yu_ref[...] = jnp.dot(xx, au_ref[...], preferred_element_type=jnp.float32) + bu_ref[...]
        yd_ref[...] = jnp.dot(xx, ad_ref[...], preferred_element_type=jnp.float32) + bd_ref[...]

    tu_ref[...] = jnp.dot(ua_ref[...], wu_ref[...], preferred_element_type=jnp.float32)
    td_ref[...] = jnp.dot(da_ref[...], wd_ref[...], preferred_element_type=jnp.float32)


def _front(x, au, bu, ad, bd, up_attr, wu, down_attr, wd):
    zero = lambda i: (0, 0)
    return pl.pallas_call(
        _front_body,
        grid=(125,),
        in_specs=[
            pl.BlockSpec((N, D), zero),
            pl.BlockSpec((D, H), zero),
            pl.BlockSpec((1, H), zero),
            pl.BlockSpec((D, H), zero),
            pl.BlockSpec((1, H), zero),
            pl.BlockSpec((BU, D), lambda i: (i, 0)),
            pl.BlockSpec((D, H), zero),
            pl.BlockSpec((BD, D), lambda i: (i, 0)),
            pl.BlockSpec((D, H), zero),
        ],
        out_specs=[
            pl.BlockSpec((N, H), zero),
            pl.BlockSpec((N, H), zero),
            pl.BlockSpec((BU, H), lambda i: (i, 0)),
            pl.BlockSpec((BD, H), lambda i: (i, 0)),
        ],
        out_shape=[
            jax.ShapeDtypeStruct((N, H), jnp.float32),
            jax.ShapeDtypeStruct((N, H), jnp.float32),
            jax.ShapeDtypeStruct((E_UP, H), jnp.float32),
            jax.ShapeDtypeStruct((E_DOWN, H), jnp.float32),
        ],
    )(x, au, bu.reshape(1, H), ad, bd.reshape(1, H), up_attr, wu, down_attr, wd)


def _sc_body(yu, tu, u0, u1, yd, td, d0, d1, x, b0, b1,
             out_up, out_dn, out_b,
             acc, tb, gb, i0b, i1b, in_sem, g_sem, sc_sem, i1_sem):
    c = lax.axis_index("c")
    s = lax.axis_index("s")

    def rows_copy(src, dst):
        r = pl.ds(pl.multiple_of(s * RPT, 8), RPT)
        pltpu.sync_copy(src.at[r], dst.at[r])

        @pl.when(s == 0)
        def _():
            rt = pl.ds(RPT * NS, RTAIL)
            pltpu.sync_copy(src.at[rt], dst.at[rt])

    def init_acc():
        rows_copy(x, acc)

    def edge_loop(y_ref, t_ref, i0_ref, i1_ref, edges_per_tile, with_t):
        n = edges_per_tile // C
        tile_base = s * edges_per_tile

        def issue_in(jj, s4):
            b = pl.multiple_of(tile_base + jj * C, 8)
            pltpu.async_copy(i0_ref.at[pl.ds(b, C)], i0b.at[s4], in_sem.at[s4])
            if with_t:
                pltpu.async_copy(t_ref.at[pl.ds(b, C)], tb.at[s4], in_sem.at[s4])

        def issue_gather_i1(jj, s4):
            pltpu.make_async_copy(i0_ref.at[pl.ds(0, C)], i0b.at[s4], in_sem.at[s4]).wait()
            pltpu.async_copy(y_ref.at[i0b.at[s4]], gb.at[s4], g_sem.at[s4])
            b = pl.multiple_of(tile_base + jj * C, 8)
            pltpu.async_copy(i1_ref.at[pl.ds(b, C)], i1b.at[s4], i1_sem.at[s4])

        def chunk(j, s4, wait_sc, pre_g, pre_in):
            s4pp = (s4 + 2) % 4
            if wait_sc:
                pltpu.make_async_copy(gb.at[s4pp], acc.at[i1b.at[s4pp]], sc_sem.at[s4pp]).wait()
            if pre_g:
                issue_gather_i1(j + 2, s4pp)
            if pre_in:
                issue_in(j + 3, (s4 + 3) % 4)
            pltpu.make_async_copy(y_ref.at[i0b.at[s4]], gb.at[s4], g_sem.at[s4]).wait()
            if with_t:
                pltpu.make_async_copy(t_ref.at[pl.ds(0, C)], tb.at[s4], in_sem.at[s4]).wait()
                gs = gb.at[s4]
                ts = tb.at[s4]

                @plsc.parallel_loop(0, C, 1, unroll=8)
                def _(r):
                    for cb in range(H // L):
                        sl = pl.ds(cb * L, L)
                        gs[r, sl] = jnp.maximum(gs[r, sl] + ts[r, sl], 0.0)
            pltpu.make_async_copy(i1_ref.at[pl.ds(0, C)], i1b.at[s4], i1_sem.at[s4]).wait()
            pltpu.async_copy(gb.at[s4], acc.at[i1b.at[s4]], sc_sem.at[s4], add=True)

        issue_in(0, 0)
        issue_in(1, 1)
        issue_in(2, 2)
        issue_gather_i1(0, 0)
        issue_gather_i1(1, 1)
        for j in range(4):
            chunk(j, j % 4, j >= 2, j + 2 < n, j + 3 < n)
        g4 = n // 4

        def group(jg, carry):
            for r in range(4):
                chunk(jg * 4 + r, r, True, True, True)
            return carry

        lax.fori_loop(1, g4 - 1, group, 0, unroll=False)
        for j in range(4 * (g4 - 1), n):
            chunk(j, j % 4, j >= 2, j + 2 < n, j + 3 < n)
        for j in range(n - 2, n):
            s4 = j % 4
            pltpu.make_async_copy(gb.at[s4], acc.at[i1b.at[s4]], sc_sem.at[s4]).wait()

    init_acc()
    plsc.subcore_barrier()

    @pl.when(c == 0)
    def _():
        edge_loop(yu, tu, u0, u1, E_UP // NS, True)

    @pl.when(c == 1)
    def _():
        edge_loop(yd, td, d0, d1, E_DOWN // NS, True)

    plsc.subcore_barrier()

    @pl.when(c == 0)
    def _():
        rows_copy(acc, out_up)

    @pl.when(c == 1)
    def _():
        rows_copy(acc, out_dn)

    plsc.subcore_barrier()

    @pl.when(c == 1)
    def _():
        init_acc()

    plsc.subcore_barrier()

    @pl.when(c == 1)
    def _():
        edge_loop(x, None, b0, b1, E_B // NS, False)

    plsc.subcore_barrier()

    @pl.when(c == 1)
    def _():
        rows_copy(acc, out_b)


def _sc_aggregate(yu, tu, u0, u1, yd, td, d0, d1, x, b0, b1):
    mesh = plsc.VectorSubcoreMesh(
        core_axis_name="c", subcore_axis_name="s", num_cores=NC, num_subcores=NS)
    return pl.kernel(
        _sc_body,
        out_type=[jax.ShapeDtypeStruct((N, H), jnp.float32)] * 3,
        mesh=mesh,
        scratch_types=[
            pltpu.VMEM_SHARED((N, H), jnp.float32),
            pltpu.VMEM((4, C, H), jnp.float32),
            pltpu.VMEM((4, C, H), jnp.float32),
            pltpu.VMEM((4, C), jnp.int32),
            pltpu.VMEM((4, C), jnp.int32),
            pltpu.SemaphoreType.DMA((4,)),
            pltpu.SemaphoreType.DMA((4,)),
            pltpu.SemaphoreType.DMA((4,)),
            pltpu.SemaphoreType.DMA((4,)),
        ],
    )(yu, tu, u0, u1, yd, td, d0, d1, x, b0, b1)


def _bn_relu(h, g, b):
    mu = jnp.mean(h, axis=0, keepdims=True)
    var = jnp.mean((h - mu) * (h - mu), axis=0, keepdims=True)
    return jnp.maximum((h - mu) * lax.rsqrt(var + 1e-5) * g + b, 0.0)


def _update_path(h, w1, b1, g1, be1, w2, b2, g2, be2):
    h = _bn_relu(jnp.dot(h, w1, preferred_element_type=jnp.float32) + b1, g1, be1)
    h = _bn_relu(jnp.dot(h, w2, preferred_element_type=jnp.float32) + b2, g2, be2)
    return h


def _final_body(au_ref, ad_ref, ab_ref, *refs):
    prefs = refs[:-1]
    o_ref = refs[-1]
    pu = [r[...] for r in prefs[0:8]]
    pd = [r[...] for r in prefs[8:16]]
    pb = [r[...] for r in prefs[16:24]]
    wc1, wc2, wc3, bc, gc, bec = [r[...] for r in prefs[24:30]]
    hu = _update_path(au_ref[...], *pu)
    hd = _update_path(ad_ref[...], *pd)
    hb = _update_path(ab_ref[...], *pb)
    h = (jnp.dot(hu, wc1, preferred_element_type=jnp.float32)
         + jnp.dot(hd, wc2, preferred_element_type=jnp.float32)
         + jnp.dot(hb, wc3, preferred_element_type=jnp.float32)) + bc
    o_ref[...] = _bn_relu(h, gc, bec)


def _final(agg_u, agg_d, agg_b, pu, pd, pb, wc, bc, gc, bec):
    def flat(p):
        w1, b1, g1, be1, w2, b2, g2, be2 = p
        return [w1, b1.reshape(1, H), g1.reshape(1, H), be1.reshape(1, H),
                w2, b2.reshape(1, H), g2.reshape(1, H), be2.reshape(1, H)]

    args = ([agg_u, agg_d, agg_b] + flat(pu) + flat(pd) + flat(pb)
            + [wc[0:H], wc[H:2 * H], wc[2 * H:3 * H],
               bc.reshape(1, H), gc.reshape(1, H), bec.reshape(1, H)])
    return pl.pallas_call(
        _final_body,
        out_shape=jax.ShapeDtypeStruct((N, H), jnp.float32),
    )(*args)


@jax.jit
def kernel(x, up_index, up_attr, down_index, down_attr, boundary_index, params):
    wmu, bmu, wmd, bmd, pu, pd, pb, wc, bc, gc, bec = params
    yu, yd, tu, td = _front(x, wmu[:D], bmu, wmd[:D], bmd,
                            up_attr, wmu[D:], down_attr, wmd[D:])
    agg_u, agg_d, agg_b = _sc_aggregate(
        yu, tu, up_index[0], up_index[1],
        yd, td, down_index[0], down_index[1],
        x, boundary_index[0], boundary_index[1])
    return _final(agg_u, agg_d, agg_b, pu, pd, pb, wc, bc, gc, bec)
